# baseline (speedup 1.0000x reference)
import functools

import jax
import jax.numpy as jnp
from jax import lax
from jax.experimental import pallas as pl
from jax.experimental.pallas import tpu as pltpu
from jax.experimental.pallas import tpu_sc as plsc

VOCAB = 1000000
D_MODEL = 128
CUT1 = 20000
CUT2 = 200000
EMB_SCALE = float(D_MODEL) ** 0.5

N_TOKENS = 1024 * 200
NHALF = N_TOKENS // 2
NW = 32
G = 128

TC_BLK = 4096

_WIDTHS = (128, 32, 8)


def _sc_route_gather_fn(nt):
    bw = nt // NW
    nch = bw // G
    vpw = bw // 16
    trash = nt

    mesh = plsc.VectorSubcoreMesh(core_axis_name="c", subcore_axis_name="s")

    scratch = [pltpu.VMEM((bw,), jnp.int32)]
    for _ in range(3):
        scratch += [
            pltpu.VMEM((nch, G), jnp.int32),
            pltpu.VMEM((nch, G), jnp.int32),
        ]
    for w in _WIDTHS:
        scratch += [
            pltpu.VMEM((G, w), jnp.float32),
            pltpu.VMEM((G, w), jnp.float32),
            pltpu.SemaphoreType.DMA,
            pltpu.SemaphoreType.DMA,
        ]

    @functools.partial(
        pl.kernel,
        out_type=(
            jax.ShapeDtypeStruct((nt + 8, 128), jnp.float32),
            jax.ShapeDtypeStruct((nt + 8, 32), jnp.float32),
            jax.ShapeDtypeStruct((nt + 8, 8), jnp.float32),
        ),
        mesh=mesh,
        compiler_params=pltpu.CompilerParams(
            use_tc_tiling_on_sc=False, needs_layout_passes=False),
        scratch_types=scratch,
    )
    def sc_fn(ids_hbm, t0_hbm, t1_hbm, t2_hbm, g0_hbm, g1_hbm, g2_hbm,
              ids_v,
              idx0_v, pos0_v, idx1_v, pos1_v, idx2_v, pos2_v,
              b0a, b0b, s0a, s0b, b1a, b1b, s1a, s1b, b2a, b2b, s2a, s2b):
        wid = lax.axis_index("s") * 2 + lax.axis_index("c")
        base = wid * bw
        pltpu.sync_copy(ids_hbm.at[pl.ds(base, bw)], ids_v)

        idx_refs = (idx0_v, idx1_v, idx2_v)
        pos_refs = (pos0_v, pos1_v, pos2_v)

        zeros = jnp.zeros((16,), jnp.int32)
        trash_v = jnp.full((16,), trash, jnp.int32)

        def init_body(i, _):
            r = i >> 3
            col = (i & 7) * 16
            for c in range(3):
                idx_refs[c][r, pl.ds(col, 16)] = zeros
                pos_refs[c][r, pl.ds(col, 16)] = trash_v
            return 0

        lax.fori_loop(0, vpw, init_body, 0, unroll=4)

        lanes = lax.iota(jnp.int32, 16)

        def route_body(i, carry):
            n0, n1, n2 = carry
            v = ids_v[pl.ds(i * 16, 16)]
            pos = (base + i * 16) + lanes
            m0 = v < CUT1
            m2 = v >= CUT2
            m1 = (v >= CUT1) & (v < CUT2)
            outs = []
            for c, (m, loc, n) in enumerate((
                    (m0, v, n0),
                    (m1, v - CUT1, n1),
                    (m2, v - CUT2, n2))):
                mc = m.astype(jnp.int32)
                tgt = n + plsc.cumsum(mc) - mc
                row = lax.shift_right_logical(tgt, 7)
                col = lax.bitwise_and(tgt, 127)
                plsc.store_scatter(idx_refs[c], [row, col], loc, mask=m)
                plsc.store_scatter(pos_refs[c], [row, col], pos, mask=m)
                outs.append(n + jnp.sum(mc))
            return tuple(outs)

        n0, n1, n2 = lax.fori_loop(
            0, vpw, route_body,
            (jnp.int32(0), jnp.int32(0), jnp.int32(0)))

        for t_hbm, g_hbm, idx_r, pos_r, bufs, sems, n in (
                (t0_hbm, g0_hbm, idx0_v, pos0_v, (b0a, b0b), (s0a, s0b), n0),
                (t1_hbm, g1_hbm, idx1_v, pos1_v, (b1a, b1b), (s1a, s1b), n1),
                (t2_hbm, g2_hbm, idx2_v, pos2_v, (b2a, b2b), (s2a, s2b), n2)):
            k = lax.shift_right_logical(n + (G - 1), 7)

            def start(j, b, t_hbm=t_hbm, idx_r=idx_r, bufs=bufs, sems=sems):
                pltpu.async_copy(t_hbm.at[idx_r.at[j]], bufs[b], sems[b])

            def drain_scatter(j, b, t_hbm=t_hbm, g_hbm=g_hbm, idx_r=idx_r,
                              pos_r=pos_r, bufs=bufs, sems=sems):
                pltpu.make_async_copy(
                    t_hbm.at[idx_r.at[0]], bufs[b], sems[b]).wait()
                pltpu.sync_copy(bufs[b], g_hbm.at[pos_r.at[j]])

            @pl.when(k > 0)
            def _():
                start(0, 0)

            @pl.when(k > 1)
            def _():
                start(1, 1)

            def pair_body(jo, _, k=k, start=start, drain_scatter=drain_scatter):
                j = jo * 2

                @pl.when(j < k)
                def _():
                    drain_scatter(j, 0)

                    @pl.when(j + 2 < k)
                    def _():
                        start(j + 2, 0)

                @pl.when(j + 1 < k)
                def _():
                    drain_scatter(j + 1, 1)

                    @pl.when(j + 3 < k)
                    def _():
                        start(j + 3, 1)

                return 0

            lax.fori_loop(0, lax.shift_right_logical(k + 1, 1), pair_body, 0)

    return sc_fn


_SC_CACHE = {}


def _sc_route_gather(ids_flat, table0, table1, table2):
    nt = ids_flat.shape[0]
    if nt not in _SC_CACHE:
        _SC_CACHE[nt] = _sc_route_gather_fn(nt)
    return _SC_CACHE[nt](ids_flat, table0, table1, table2)


def _tc_compute(ids_ref, g0_ref, g1_ref, g2_ref,
                p0_ref, p1_ref, p2_ref, out_ref):
    ids = ids_ref[...]
    dn = (((1,), (1,)), ((), ()))
    o0 = lax.dot_general(g0_ref[...], p0_ref[...], dn,
                         preferred_element_type=jnp.float32)
    o1 = lax.dot_general(g1_ref[...], p1_ref[...], dn,
                         preferred_element_type=jnp.float32)
    o2 = lax.dot_general(g2_ref[...], p2_ref[...], dn,
                         preferred_element_type=jnp.float32)
    m0 = ids < CUT1
    m1 = ids < CUT2
    out = jnp.where(m0, o0, jnp.where(m1, o1, o2))
    out_ref[...] = out * EMB_SCALE


def _tc_body_first(ids_ref, g0_ref, g1_ref, g2_ref,
                   p0_ref, p1_ref, p2_ref, out_ref):
    _tc_compute(ids_ref, g0_ref, g1_ref, g2_ref,
                p0_ref, p1_ref, p2_ref, out_ref)


def _tc_body_second(prev_ref, ids_ref, g0_ref, g1_ref, g2_ref,
                    p0_ref, p1_ref, p2_ref, out_ref):
    del prev_ref
    _tc_compute(ids_ref, g0_ref, g1_ref, g2_ref,
                p0_ref, p1_ref, p2_ref, out_ref)


_HALF_BLOCKS = NHALF // TC_BLK


def _tc_first(ids_col, g0, g1, g2, proj0, proj1, proj2):
    return pl.pallas_call(
        _tc_body_first,
        grid=(_HALF_BLOCKS,),
        in_specs=[
            pl.BlockSpec((TC_BLK, 1), lambda i: (i, 0)),
            pl.BlockSpec((TC_BLK, 128), lambda i: (i, 0)),
            pl.BlockSpec((TC_BLK, 32), lambda i: (i, 0)),
            pl.BlockSpec((TC_BLK, 8), lambda i: (i, 0)),
            pl.BlockSpec((128, 128), lambda i: (0, 0)),
            pl.BlockSpec((128, 32), lambda i: (0, 0)),
            pl.BlockSpec((128, 8), lambda i: (0, 0)),
        ],
        out_specs=pl.BlockSpec((TC_BLK, D_MODEL), lambda i: (i, 0)),
        out_shape=jax.ShapeDtypeStruct((N_TOKENS, D_MODEL), jnp.float32),
    )(ids_col, g0, g1, g2, proj0, proj1, proj2)


def _tc_second(prev_out, ids_col, g0, g1, g2, proj0, proj1, proj2):
    return pl.pallas_call(
        _tc_body_second,
        grid=(_HALF_BLOCKS,),
        in_specs=[
            pl.BlockSpec(memory_space=pl.ANY),
            pl.BlockSpec((TC_BLK, 1), lambda i: (i, 0)),
            pl.BlockSpec((TC_BLK, 128), lambda i: (i, 0)),
            pl.BlockSpec((TC_BLK, 32), lambda i: (i, 0)),
            pl.BlockSpec((TC_BLK, 8), lambda i: (i, 0)),
            pl.BlockSpec((128, 128), lambda i: (0, 0)),
            pl.BlockSpec((128, 32), lambda i: (0, 0)),
            pl.BlockSpec((128, 8), lambda i: (0, 0)),
        ],
        out_specs=pl.BlockSpec((TC_BLK, D_MODEL),
                               lambda i: (i + _HALF_BLOCKS, 0)),
        out_shape=jax.ShapeDtypeStruct((N_TOKENS, D_MODEL), jnp.float32),
        input_output_aliases={0: 0},
    )(prev_out, ids_col, g0, g1, g2, proj0, proj1, proj2)


def kernel(input_ids, table0, table1, table2, proj0, proj1, proj2):
    ids_flat = input_ids.reshape(-1)
    ids_a = ids_flat[:NHALF]
    ids_b = ids_flat[NHALF:]
    ga = _sc_route_gather(ids_a, table0, table1, table2)
    gb = _sc_route_gather(ids_b, table0, table1, table2)
    out = _tc_first(ids_a.reshape(-1, 1), ga[0], ga[1], ga[2],
                    proj0, proj1, proj2)
    out = _tc_second(out, ids_b.reshape(-1, 1), gb[0], gb[1], gb[2],
                     proj0, proj1, proj2)
    return out.reshape(input_ids.shape + (D_MODEL,))


# speedup vs baseline: 1.1363x; 1.1363x over previous
import functools

import jax
import jax.numpy as jnp
from jax import lax
from jax.experimental import pallas as pl
from jax.experimental.pallas import tpu as pltpu
from jax.experimental.pallas import tpu_sc as plsc

VOCAB = 1000000
D_MODEL = 128
CUT1 = 20000
CUT2 = 200000
EMB_SCALE = float(D_MODEL) ** 0.5

N_TOKENS = 1024 * 200
NW = 32
BW = N_TOKENS // NW
G = 128
NCH = BW // G
VPW = BW // 16
TRASH = N_TOKENS

TC_BLK = 4096

_WIDTHS = (128, 32, 8)


def _sc_route_gather_fn():
    mesh = plsc.VectorSubcoreMesh(core_axis_name="c", subcore_axis_name="s")

    scratch = [pltpu.VMEM((BW,), jnp.int32)]
    for _ in range(3):
        scratch += [
            pltpu.VMEM((NCH, G), jnp.int32),
            pltpu.VMEM((NCH, G), jnp.int32),
        ]
    for w in _WIDTHS:
        scratch += [
            pltpu.VMEM((G, w), jnp.float32),
            pltpu.VMEM((G, w), jnp.float32),
            pltpu.SemaphoreType.DMA,
            pltpu.SemaphoreType.DMA,
        ]

    @functools.partial(
        pl.kernel,
        out_type=(
            jax.ShapeDtypeStruct((N_TOKENS + 8, 128), jnp.float32),
            jax.ShapeDtypeStruct((N_TOKENS + 8, 32), jnp.float32),
            jax.ShapeDtypeStruct((N_TOKENS + 8, 8), jnp.float32),
        ),
        mesh=mesh,
        compiler_params=pltpu.CompilerParams(use_tc_tiling_on_sc=False, needs_layout_passes=False),
        scratch_types=scratch,
    )
    def sc_fn(ids_hbm, t0_hbm, t1_hbm, t2_hbm, g0_hbm, g1_hbm, g2_hbm,
              ids_v,
              idx0_v, pos0_v, idx1_v, pos1_v, idx2_v, pos2_v,
              b0a, b0b, s0a, s0b, b1a, b1b, s1a, s1b, b2a, b2b, s2a, s2b):
        wid = lax.axis_index("s") * 2 + lax.axis_index("c")
        base = wid * BW
        pltpu.sync_copy(ids_hbm.at[pl.ds(base, BW)], ids_v)

        idx_refs = (idx0_v, idx1_v, idx2_v)
        pos_refs = (pos0_v, pos1_v, pos2_v)

        zeros = jnp.zeros((16,), jnp.int32)
        trash = jnp.full((16,), TRASH, jnp.int32)

        def init_body(i, _):
            r = i >> 3
            col = (i & 7) * 16
            for c in range(3):
                idx_refs[c][r, pl.ds(col, 16)] = zeros
                pos_refs[c][r, pl.ds(col, 16)] = trash
            return 0

        lax.fori_loop(0, VPW, init_body, 0, unroll=4)

        lanes = lax.iota(jnp.int32, 16)

        def route_body(i, carry):
            n0, n1, n2 = carry
            v = ids_v[pl.ds(i * 16, 16)]
            pos = (base + i * 16) + lanes
            m0 = v < CUT1
            m2 = v >= CUT2
            m1 = (v >= CUT1) & (v < CUT2)
            outs = []
            for c, (m, loc, n) in enumerate((
                    (m0, v, n0),
                    (m1, v - CUT1, n1),
                    (m2, v - CUT2, n2))):
                mc = m.astype(jnp.int32)
                tgt = n + plsc.cumsum(mc) - mc
                row = lax.shift_right_logical(tgt, 7)
                col = lax.bitwise_and(tgt, 127)
                plsc.store_scatter(idx_refs[c], [row, col], loc, mask=m)
                plsc.store_scatter(pos_refs[c], [row, col], pos, mask=m)
                outs.append(n + jnp.sum(mc))
            return tuple(outs)

        n0, n1, n2 = lax.fori_loop(
            0, VPW, route_body,
            (jnp.int32(0), jnp.int32(0), jnp.int32(0)))

        for t_hbm, g_hbm, idx_r, pos_r, bufs, sems, n in (
                (t0_hbm, g0_hbm, idx0_v, pos0_v, (b0a, b0b), (s0a, s0b), n0),
                (t1_hbm, g1_hbm, idx1_v, pos1_v, (b1a, b1b), (s1a, s1b), n1),
                (t2_hbm, g2_hbm, idx2_v, pos2_v, (b2a, b2b), (s2a, s2b), n2)):
            k = lax.shift_right_logical(n + (G - 1), 7)

            def start(j, b, t_hbm=t_hbm, idx_r=idx_r, bufs=bufs, sems=sems):
                pltpu.async_copy(t_hbm.at[idx_r.at[j]], bufs[b], sems[b])

            def drain_scatter(j, b, t_hbm=t_hbm, g_hbm=g_hbm, idx_r=idx_r,
                              pos_r=pos_r, bufs=bufs, sems=sems):
                pltpu.make_async_copy(
                    t_hbm.at[idx_r.at[0]], bufs[b], sems[b]).wait()
                pltpu.sync_copy(bufs[b], g_hbm.at[pos_r.at[j]])

            @pl.when(k > 0)
            def _():
                start(0, 0)

            @pl.when(k > 1)
            def _():
                start(1, 1)

            def pair_body(jo, _, k=k, start=start, drain_scatter=drain_scatter):
                j = jo * 2

                @pl.when(j < k)
                def _():
                    drain_scatter(j, 0)

                    @pl.when(j + 2 < k)
                    def _():
                        start(j + 2, 0)

                @pl.when(j + 1 < k)
                def _():
                    drain_scatter(j + 1, 1)

                    @pl.when(j + 3 < k)
                    def _():
                        start(j + 3, 1)

                return 0

            lax.fori_loop(0, lax.shift_right_logical(k + 1, 1), pair_body, 0)

    return sc_fn


_SC_CACHE = {}


def _sc_route_gather(ids_flat, table0, table1, table2):
    if "fn" not in _SC_CACHE:
        _SC_CACHE["fn"] = _sc_route_gather_fn()
    return _SC_CACHE["fn"](ids_flat, table0, table1, table2)


def _tc_body(ids_ref, g0_ref, g1_ref, g2_ref, p0_ref, p1_ref, p2_ref, out_ref):
    ids = ids_ref[...]
    dn = (((1,), (1,)), ((), ()))
    m0 = ids < CUT1
    m1 = (ids >= CUT1) & (ids < CUT2)
    m2 = ids >= CUT2
    o0 = lax.dot_general(jnp.where(m0, g0_ref[...], 0.0), p0_ref[...], dn,
                         preferred_element_type=jnp.float32)
    o1 = lax.dot_general(jnp.where(m1, g1_ref[...], 0.0), p1_ref[...], dn,
                         preferred_element_type=jnp.float32)
    o2 = lax.dot_general(jnp.where(m2, g2_ref[...], 0.0), p2_ref[...], dn,
                         preferred_element_type=jnp.float32)
    out_ref[...] = (o0 + o1 + o2) * EMB_SCALE


def _tc_combine(ids_col, g0, g1, g2, proj0, proj1, proj2):
    grid = (N_TOKENS // TC_BLK,)
    return pl.pallas_call(
        _tc_body,
        grid=grid,
        in_specs=[
            pl.BlockSpec((TC_BLK, 1), lambda i: (i, 0)),
            pl.BlockSpec((TC_BLK, 128), lambda i: (i, 0)),
            pl.BlockSpec((TC_BLK, 32), lambda i: (i, 0)),
            pl.BlockSpec((TC_BLK, 8), lambda i: (i, 0)),
            pl.BlockSpec((128, 128), lambda i: (0, 0)),
            pl.BlockSpec((128, 32), lambda i: (0, 0)),
            pl.BlockSpec((128, 8), lambda i: (0, 0)),
        ],
        out_specs=pl.BlockSpec((TC_BLK, D_MODEL), lambda i: (i, 0)),
        out_shape=jax.ShapeDtypeStruct((N_TOKENS, D_MODEL), jnp.float32),
    )(ids_col, g0, g1, g2, proj0, proj1, proj2)


def kernel(input_ids, table0, table1, table2, proj0, proj1, proj2):
    ids_flat = input_ids.reshape(-1)
    g0, g1, g2 = _sc_route_gather(ids_flat, table0, table1, table2)
    out = _tc_combine(ids_flat.reshape(-1, 1), g0, g1, g2, proj0, proj1, proj2)
    return out.reshape(input_ids.shape + (D_MODEL,))


# speedup vs baseline: 1.1503x; 1.0123x over previous
import functools

import jax
import jax.numpy as jnp
from jax import lax
from jax.experimental import pallas as pl
from jax.experimental.pallas import tpu as pltpu
from jax.experimental.pallas import tpu_sc as plsc

VOCAB = 1000000
D_MODEL = 128
CUT1 = 20000
CUT2 = 200000
EMB_SCALE = float(D_MODEL) ** 0.5

N_TOKENS = 1024 * 200
NW = 32
BW = N_TOKENS // NW
G = 128
NCH = BW // G
VPW = BW // 16
TRASH = N_TOKENS

TC_BLK = 4096

_WIDTHS = (128, 32, 8)


def _sc_route_gather_fn():
    mesh = plsc.VectorSubcoreMesh(core_axis_name="c", subcore_axis_name="s")

    scratch = [pltpu.VMEM((BW,), jnp.int32)]
    for _ in range(3):
        scratch += [
            pltpu.VMEM((NCH, G), jnp.int32),
            pltpu.VMEM((NCH, G), jnp.int32),
        ]
    for w in _WIDTHS:
        scratch += [
            pltpu.VMEM((G, w), jnp.float32),
            pltpu.VMEM((G, w), jnp.float32),
            pltpu.SemaphoreType.DMA,
            pltpu.SemaphoreType.DMA,
        ]

    @functools.partial(
        pl.kernel,
        out_type=(
            jax.ShapeDtypeStruct((N_TOKENS + 8, 128), jnp.float32),
            jax.ShapeDtypeStruct((N_TOKENS + 8, 32), jnp.float32),
            jax.ShapeDtypeStruct((N_TOKENS + 8, 8), jnp.float32),
        ),
        mesh=mesh,
        compiler_params=pltpu.CompilerParams(use_tc_tiling_on_sc=False, needs_layout_passes=False),
        scratch_types=scratch,
    )
    def sc_fn(ids_hbm, t0_hbm, t1_hbm, t2_hbm, g0_hbm, g1_hbm, g2_hbm,
              ids_v,
              idx0_v, pos0_v, idx1_v, pos1_v, idx2_v, pos2_v,
              b0a, b0b, s0a, s0b, b1a, b1b, s1a, s1b, b2a, b2b, s2a, s2b):
        wid = lax.axis_index("s") * 2 + lax.axis_index("c")
        base = wid * BW
        idq = BW // 4
        ld = []
        for q, sem in enumerate((s0a, s0b, s1a, s1b)):
            ld.append(pltpu.async_copy(
                ids_hbm.at[pl.ds(base + q * idq, idq)],
                ids_v.at[pl.ds(q * idq, idq)], sem))
        for c in ld:
            c.wait()

        idx_refs = (idx0_v, idx1_v, idx2_v)
        pos_refs = (pos0_v, pos1_v, pos2_v)

        zeros = jnp.zeros((16,), jnp.int32)
        trash = jnp.full((16,), TRASH, jnp.int32)

        def init_body(i, _):
            r = i >> 3
            col = (i & 7) * 16
            for c in range(3):
                idx_refs[c][r, pl.ds(col, 16)] = zeros
                pos_refs[c][r, pl.ds(col, 16)] = trash
            return 0

        lax.fori_loop(0, VPW, init_body, 0, unroll=4)

        lanes = lax.iota(jnp.int32, 16)

        def route_body(i, carry):
            n0, n1, n2 = carry
            v = ids_v[pl.ds(i * 16, 16)]
            pos = (base + i * 16) + lanes
            m0 = v < CUT1
            m2 = v >= CUT2
            m1 = (v >= CUT1) & (v < CUT2)
            outs = []
            for c, (m, loc, n) in enumerate((
                    (m0, v, n0),
                    (m1, v - CUT1, n1),
                    (m2, v - CUT2, n2))):
                mc = m.astype(jnp.int32)
                tgt = n + plsc.cumsum(mc) - mc
                row = lax.shift_right_logical(tgt, 7)
                col = lax.bitwise_and(tgt, 127)
                plsc.store_scatter(idx_refs[c], [row, col], loc, mask=m)
                plsc.store_scatter(pos_refs[c], [row, col], pos, mask=m)
                outs.append(n + plsc.all_reduce_population_count(m))
            return tuple(outs)

        zero_v = jnp.zeros((16,), jnp.int32)
        n0_v, n1_v, n2_v = lax.fori_loop(
            0, VPW, route_body, (zero_v, zero_v, zero_v), unroll=2)
        n0 = jnp.max(n0_v)
        n1 = jnp.max(n1_v)
        n2 = jnp.max(n2_v)

        for t_hbm, g_hbm, idx_r, pos_r, bufs, sems, n in (
                (t0_hbm, g0_hbm, idx0_v, pos0_v, (b0a, b0b), (s0a, s0b), n0),
                (t1_hbm, g1_hbm, idx1_v, pos1_v, (b1a, b1b), (s1a, s1b), n1),
                (t2_hbm, g2_hbm, idx2_v, pos2_v, (b2a, b2b), (s2a, s2b), n2)):
            k = lax.shift_right_logical(n + (G - 1), 7)

            def start(j, b, t_hbm=t_hbm, idx_r=idx_r, bufs=bufs, sems=sems):
                pltpu.async_copy(t_hbm.at[idx_r.at[j]], bufs[b], sems[b])

            def drain_scatter(j, b, t_hbm=t_hbm, g_hbm=g_hbm, idx_r=idx_r,
                              pos_r=pos_r, bufs=bufs, sems=sems):
                pltpu.make_async_copy(
                    t_hbm.at[idx_r.at[0]], bufs[b], sems[b]).wait()
                pltpu.sync_copy(bufs[b], g_hbm.at[pos_r.at[j]])

            @pl.when(k > 0)
            def _():
                start(0, 0)

            @pl.when(k > 1)
            def _():
                start(1, 1)

            def pair_body(jo, _, k=k, start=start, drain_scatter=drain_scatter):
                j = jo * 2

                @pl.when(j < k)
                def _():
                    drain_scatter(j, 0)

                    @pl.when(j + 2 < k)
                    def _():
                        start(j + 2, 0)

                @pl.when(j + 1 < k)
                def _():
                    drain_scatter(j + 1, 1)

                    @pl.when(j + 3 < k)
                    def _():
                        start(j + 3, 1)

                return 0

            lax.fori_loop(0, lax.shift_right_logical(k + 1, 1), pair_body, 0)

    return sc_fn


_SC_CACHE = {}


def _sc_route_gather(ids_flat, table0, table1, table2):
    if "fn" not in _SC_CACHE:
        _SC_CACHE["fn"] = _sc_route_gather_fn()
    return _SC_CACHE["fn"](ids_flat, table0, table1, table2)


def _tc_body(ids_ref, g0_ref, g1_ref, g2_ref, p0_ref, p1_ref, p2_ref, out_ref):
    ids = ids_ref[...]
    dn = (((1,), (1,)), ((), ()))
    o0 = lax.dot_general(g0_ref[...], p0_ref[...], dn,
                         preferred_element_type=jnp.float32)
    o1 = lax.dot_general(g1_ref[...], p1_ref[...], dn,
                         preferred_element_type=jnp.float32)
    o2 = lax.dot_general(g2_ref[...], p2_ref[...], dn,
                         preferred_element_type=jnp.float32)
    m0 = ids < CUT1
    m1 = ids < CUT2
    out = jnp.where(m0, o0, jnp.where(m1, o1, o2))
    out_ref[...] = out * EMB_SCALE


def _tc_combine(ids_col, g0, g1, g2, proj0, proj1, proj2):
    grid = (N_TOKENS // TC_BLK,)
    return pl.pallas_call(
        _tc_body,
        grid=grid,
        in_specs=[
            pl.BlockSpec((TC_BLK, 1), lambda i: (i, 0)),
            pl.BlockSpec((TC_BLK, 128), lambda i: (i, 0)),
            pl.BlockSpec((TC_BLK, 32), lambda i: (i, 0)),
            pl.BlockSpec((TC_BLK, 8), lambda i: (i, 0)),
            pl.BlockSpec((128, 128), lambda i: (0, 0)),
            pl.BlockSpec((128, 32), lambda i: (0, 0)),
            pl.BlockSpec((128, 8), lambda i: (0, 0)),
        ],
        out_specs=pl.BlockSpec((TC_BLK, D_MODEL), lambda i: (i, 0)),
        out_shape=jax.ShapeDtypeStruct((N_TOKENS, D_MODEL), jnp.float32),
    )(ids_col, g0, g1, g2, proj0, proj1, proj2)


def kernel(input_ids, table0, table1, table2, proj0, proj1, proj2):
    ids_flat = input_ids.reshape(-1)
    g0, g1, g2 = _sc_route_gather(ids_flat, table0, table1, table2)
    out = _tc_combine(ids_flat.reshape(-1, 1), g0, g1, g2, proj0, proj1, proj2)
    return out.reshape(input_ids.shape + (D_MODEL,))
